# baseline (device time: 376722 ns/iter reference)
import jax
import jax.numpy as jnp
from jax import lax
from jax.experimental import pallas as pl
from jax.experimental.pallas import tpu as pltpu

N_DEV = 8
N_TOK = 2048
D_MODEL = 512
D_HID = 1024
N_EXP = 32
E_LOCAL = N_EXP // N_DEV
CAP = 51


def kernel(x, router_W, route_idx, expert_W):
    e_ids = route_idx[:, 0]
    onehot = (e_ids[:, None] == jnp.arange(N_EXP, dtype=e_ids.dtype)[None, :])
    pos = jnp.cumsum(onehot.astype(jnp.int32), axis=0)
    keep = onehot & (pos <= CAP)
    my = lax.axis_index("i")
    local_keep = lax.dynamic_slice(
        keep.astype(jnp.bfloat16), (0, my * E_LOCAL), (N_TOK, E_LOCAL)
    )

    xb = x.astype(jnp.bfloat16)
    wb = expert_W.astype(jnp.bfloat16)

    def body(x_ref, mask_ref, w_ref, out_ref, comm_ref, send_sems, recv_sems,
             credit_sem):
        my_pos = lax.axis_index("i")
        left = lax.rem(my_pos - 1 + N_DEV, N_DEV)
        right = lax.rem(my_pos + 1, N_DEV)

        barrier_sem = pltpu.get_barrier_semaphore()
        for nbr in [left, right]:
            pl.semaphore_signal(
                barrier_sem, inc=1,
                device_id=(nbr,), device_id_type=pl.DeviceIdType.MESH,
            )
        pl.semaphore_wait(barrier_sem, 2)

        acc = jnp.zeros((N_TOK, D_HID), jnp.float32)
        for le in range(E_LOCAL):
            xm = x_ref[:, :] * mask_ref[:, le][:, None]
            acc = acc + jnp.dot(
                xm, w_ref[le], preferred_element_type=jnp.float32
            )
        comm_ref[0] = acc.astype(jnp.bfloat16)

        for h in range(N_DEV - 1):
            send_slot = h % 2
            recv_slot = (h + 1) % 2
            if h >= 1:
                pl.semaphore_wait(credit_sem, 1)
            rdma = pltpu.make_async_remote_copy(
                src_ref=comm_ref.at[send_slot],
                dst_ref=comm_ref.at[recv_slot],
                send_sem=send_sems.at[send_slot],
                recv_sem=recv_sems.at[recv_slot],
                device_id=(right,),
                device_id_type=pl.DeviceIdType.MESH,
            )
            rdma.start()
            rdma.wait()
            acc = acc + comm_ref[recv_slot][...].astype(jnp.float32)
            if h < N_DEV - 2:
                pl.semaphore_signal(
                    credit_sem, inc=1,
                    device_id=(left,), device_id_type=pl.DeviceIdType.MESH,
                )
        out_ref[:, :] = acc

    return pl.pallas_call(
        body,
        out_shape=jax.ShapeDtypeStruct((N_TOK, D_HID), jnp.float32),
        in_specs=[
            pl.BlockSpec(memory_space=pltpu.VMEM),
            pl.BlockSpec(memory_space=pltpu.VMEM),
            pl.BlockSpec(memory_space=pltpu.VMEM),
        ],
        out_specs=pl.BlockSpec(memory_space=pltpu.VMEM),
        scratch_shapes=[
            pltpu.VMEM((2, N_TOK, D_HID), jnp.bfloat16),
            pltpu.SemaphoreType.DMA((2,)),
            pltpu.SemaphoreType.DMA((2,)),
            pltpu.SemaphoreType.REGULAR,
        ],
        compiler_params=pltpu.CompilerParams(collective_id=0),
    )(xb, local_keep, wb)


# device time: 136121 ns/iter; 2.7676x vs baseline; 2.7676x over previous
import jax
import jax.numpy as jnp
from jax import lax
from jax.experimental import pallas as pl
from jax.experimental.pallas import tpu as pltpu

N_DEV = 8
N_TOK = 2048
D_MODEL = 512
D_HID = 1024
N_EXP = 32
E_LOCAL = N_EXP // N_DEV
CAP = 51
CHUNK = N_TOK // N_DEV


def kernel(x, router_W, route_idx, expert_W):
    e_ids = route_idx[:, 0]
    onehot = (e_ids[:, None] == jnp.arange(N_EXP, dtype=e_ids.dtype)[None, :])
    pos = jnp.cumsum(onehot.astype(jnp.int32), axis=0)
    keep = onehot & (pos <= CAP)
    my = lax.axis_index("i")
    local_keep = lax.dynamic_slice(
        keep.astype(jnp.bfloat16), (0, my * E_LOCAL), (N_TOK, E_LOCAL)
    )

    xb = x.astype(jnp.bfloat16)
    wb = expert_W.astype(jnp.bfloat16)

    def body(x_ref, mask_ref, w_ref, out_ref, red_ref, comm_ref,
             rs_send_sems, rs_recv_sems, ag_send_sems, ag_recv_sems,
             credit_sem):
        my_pos = lax.axis_index("i")
        left = lax.rem(my_pos - 1 + N_DEV, N_DEV)
        right = lax.rem(my_pos + 1, N_DEV)

        barrier_sem = pltpu.get_barrier_semaphore()
        for nbr in [left, right]:
            pl.semaphore_signal(
                barrier_sem, inc=1,
                device_id=(nbr,), device_id_type=pl.DeviceIdType.MESH,
            )
        pl.semaphore_wait(barrier_sem, 2)

        acc = jnp.zeros((N_TOK, D_HID), jnp.float32)
        for le in range(E_LOCAL):
            xm = x_ref[:, :] * mask_ref[:, le][:, None]
            acc = acc + jnp.dot(
                xm, w_ref[le], preferred_element_type=jnp.float32
            )
        red_ref[...] = acc.astype(jnp.bfloat16).reshape(N_DEV, CHUNK, D_HID)

        for s in range(N_DEV - 1):
            c_send = lax.rem(my_pos - s + N_DEV, N_DEV)
            c_recv = lax.rem(my_pos - s - 1 + N_DEV, N_DEV)
            if s >= 2:
                pl.semaphore_wait(credit_sem, 1)
            rdma = pltpu.make_async_remote_copy(
                src_ref=red_ref.at[c_send],
                dst_ref=comm_ref.at[s % 2],
                send_sem=rs_send_sems.at[s % 2],
                recv_sem=rs_recv_sems.at[s % 2],
                device_id=(right,),
                device_id_type=pl.DeviceIdType.MESH,
            )
            rdma.start()
            rdma.wait()
            red_ref[c_recv] = red_ref[c_recv] + comm_ref[s % 2]
            if s < N_DEV - 3:
                pl.semaphore_signal(
                    credit_sem, inc=1,
                    device_id=(left,), device_id_type=pl.DeviceIdType.MESH,
                )

        for t in range(N_DEV - 1):
            c = lax.rem(my_pos + 1 - t + N_DEV, N_DEV)
            if t >= 2:
                pl.semaphore_wait(credit_sem, 1)
            rdma = pltpu.make_async_remote_copy(
                src_ref=red_ref.at[c],
                dst_ref=red_ref.at[c],
                send_sem=ag_send_sems.at[t % 2],
                recv_sem=ag_recv_sems.at[t % 2],
                device_id=(right,),
                device_id_type=pl.DeviceIdType.MESH,
            )
            rdma.start()
            rdma.wait()
            if t < N_DEV - 3:
                pl.semaphore_signal(
                    credit_sem, inc=1,
                    device_id=(left,), device_id_type=pl.DeviceIdType.MESH,
                )

        out_ref[...] = red_ref[...].astype(jnp.float32).reshape(N_TOK, D_HID)

    return pl.pallas_call(
        body,
        out_shape=jax.ShapeDtypeStruct((N_TOK, D_HID), jnp.float32),
        in_specs=[
            pl.BlockSpec(memory_space=pltpu.VMEM),
            pl.BlockSpec(memory_space=pltpu.VMEM),
            pl.BlockSpec(memory_space=pltpu.VMEM),
        ],
        out_specs=pl.BlockSpec(memory_space=pltpu.VMEM),
        scratch_shapes=[
            pltpu.VMEM((N_DEV, CHUNK, D_HID), jnp.bfloat16),
            pltpu.VMEM((2, CHUNK, D_HID), jnp.bfloat16),
            pltpu.SemaphoreType.DMA((2,)),
            pltpu.SemaphoreType.DMA((2,)),
            pltpu.SemaphoreType.DMA((2,)),
            pltpu.SemaphoreType.DMA((2,)),
            pltpu.SemaphoreType.REGULAR,
        ],
        compiler_params=pltpu.CompilerParams(collective_id=0),
    )(xb, local_keep, wb)
